# baseline (speedup 1.0000x reference)
import functools

import jax
import jax.numpy as jnp
from jax import lax
from jax.experimental import pallas as pl
from jax.experimental.pallas import tpu as pltpu
from jax.experimental.pallas import tpu_sc as plsc

V = 1000000
D = 64
B = 16384
NEG = 20

NC = 2
NS = 16
NW = NC * NS
B_PER_W = B // NW
BLK = 128
NBLK = B_PER_W // BLK


def _dots(crows, prows, scores):
    lanes = lax.iota(jnp.int32, 16)

    def group_body(g, _):
        rid = g * 16 + lanes

        def d_body(d, acc):
            dv = jnp.broadcast_to(d, (16,))
            cv = plsc.load_gather(crows, [rid, dv])
            pv = plsc.load_gather(prows, [rid, dv])
            return acc + cv * pv

        acc = lax.fori_loop(0, D, d_body, jnp.zeros((16,), jnp.float32))
        scores[pl.ds(g * 16, 16)] = acc
        return _

    lax.fori_loop(0, BLK // 16, group_body, None)


def _sc_body(cw_hbm, xw_hbm, nT_hbm, in_hbm, out_hbm,
             pos_hbm, negT_hbm,
             cidx, pidx, crows, prows, scores, sem):
    wid = lax.axis_index("s") * NC + lax.axis_index("c")
    wbase = wid * B_PER_W

    def blk_body(sb, _):
        bb = wbase + sb * BLK
        pltpu.sync_copy(cw_hbm.at[pl.ds(bb, BLK)], cidx)
        pltpu.async_copy(in_hbm.at[cidx], crows, sem).wait()
        pltpu.sync_copy(xw_hbm.at[pl.ds(bb, BLK)], pidx)
        pltpu.async_copy(out_hbm.at[pidx], prows, sem).wait()
        _dots(crows, prows, scores)
        pltpu.sync_copy(scores, pos_hbm.at[pl.ds(bb, BLK)])

        def j_body(j, _):
            pltpu.sync_copy(nT_hbm.at[j, pl.ds(bb, BLK)], pidx)
            pltpu.async_copy(out_hbm.at[pidx], prows, sem).wait()
            _dots(crows, prows, scores)
            pltpu.sync_copy(scores, negT_hbm.at[j, pl.ds(bb, BLK)])
            return _

        lax.fori_loop(0, NEG, j_body, None)
        return _

    lax.fori_loop(0, NBLK, blk_body, None)


@jax.jit
def _sc_scores(center_words, context_words, neg_T, in_emb, out_emb):
    mesh = plsc.VectorSubcoreMesh(
        core_axis_name="c", subcore_axis_name="s", num_cores=NC, num_subcores=NS
    )
    f = pl.kernel(
        _sc_body,
        out_type=(
            jax.ShapeDtypeStruct((B,), jnp.float32),
            jax.ShapeDtypeStruct((NEG, B), jnp.float32),
        ),
        mesh=mesh,
        compiler_params=pltpu.CompilerParams(
            use_tc_tiling_on_sc=False, needs_layout_passes=False
        ),
        scratch_types=[
            pltpu.VMEM((BLK,), jnp.int32),
            pltpu.VMEM((BLK,), jnp.int32),
            pltpu.VMEM((BLK, D), jnp.float32),
            pltpu.VMEM((BLK, D), jnp.float32),
            pltpu.VMEM((BLK,), jnp.float32),
            pltpu.SemaphoreType.DMA,
        ],
    )
    return f(center_words, context_words, neg_T, in_emb, out_emb)


def _loss_body(pos_ref, neg_ref, out_ref):
    p = pos_ref[...]
    n = neg_ref[...]
    total = jnp.sum(jax.nn.log_sigmoid(p)) + jnp.sum(jax.nn.log_sigmoid(-n))
    out_ref[...] = jnp.reshape(-total / B, (1, 1))


@jax.jit
def _tc_loss(pos, neg):
    out = pl.pallas_call(
        _loss_body,
        out_shape=jax.ShapeDtypeStruct((1, 1), jnp.float32),
    )(pos.reshape(128, 128), neg.reshape(NEG * B // 128, 128))
    return out[0, 0]


def kernel(center_words, context_words, negative_words, in_emb, out_emb):
    neg_T = negative_words.T
    pos, negs = _sc_scores(center_words, context_words, neg_T, in_emb, out_emb)
    return _tc_loss(pos, negs)


# speedup vs baseline: 1.2042x; 1.2042x over previous
import jax
import jax.numpy as jnp
from jax import lax
from jax.experimental import pallas as pl
from jax.experimental.pallas import tpu as pltpu
from jax.experimental.pallas import tpu_sc as plsc

V = 1000000
D = 64
B = 16384
NEG = 20
NP = NEG + 1

NC = 2
NS = 16
NW = NC * NS
B_PER_W = B // NW
BLK = 64
NBLK = B_PER_W // BLK


def _sc_body(cw_hbm, xw_hbm, nT_hbm, in_hbm, out_hbm,
             pos_hbm, negT_hbm,
             cidx, xidx, nidx, crows, prows, scores, sem):
    wid = lax.axis_index("s") * NC + lax.axis_index("c")
    wbase = wid * B_PER_W
    lanes = lax.iota(jnp.int32, 16)

    pltpu.sync_copy(cw_hbm.at[pl.ds(wbase, B_PER_W)], cidx)
    pltpu.sync_copy(xw_hbm.at[pl.ds(wbase, B_PER_W)], xidx)
    for j in range(NEG):
        pltpu.sync_copy(nT_hbm.at[j, pl.ds(wbase, B_PER_W)], nidx.at[j])

    def blk_body(sb, _):
        off = sb * BLK
        descs = [
            pltpu.async_copy(in_hbm.at[cidx.at[pl.ds(off, BLK)]], crows, sem),
            pltpu.async_copy(out_hbm.at[xidx.at[pl.ds(off, BLK)]], prows.at[0], sem),
        ]
        for j in range(NEG):
            descs.append(
                pltpu.async_copy(
                    out_hbm.at[nidx.at[j, pl.ds(off, BLK)]], prows.at[1 + j], sem
                )
            )
        for dsc in descs:
            dsc.wait()

        def group_body(g, _):
            rid = g * 16 + lanes

            def d_body(d, accs):
                dv = jnp.broadcast_to(d, (16,))
                cv = plsc.load_gather(crows, [rid, dv])
                return tuple(
                    acc + cv * plsc.load_gather(prows, [jnp.broadcast_to(t, (16,)), rid, dv])
                    for t, acc in enumerate(accs)
                )

            accs = lax.fori_loop(
                0, D, d_body, tuple(jnp.zeros((16,), jnp.float32) for _ in range(NP))
            )
            for t in range(NP):
                scores[t, pl.ds(g * 16, 16)] = accs[t]
            return _

        lax.fori_loop(0, BLK // 16, group_body, None)

        pltpu.sync_copy(scores.at[0], pos_hbm.at[pl.ds(wbase + off, BLK)])
        for j in range(NEG):
            pltpu.sync_copy(scores.at[1 + j], negT_hbm.at[j, pl.ds(wbase + off, BLK)])
        return _

    lax.fori_loop(0, NBLK, blk_body, None)


@jax.jit
def _sc_scores(center_words, context_words, neg_T, in_emb, out_emb):
    mesh = plsc.VectorSubcoreMesh(
        core_axis_name="c", subcore_axis_name="s", num_cores=NC, num_subcores=NS
    )
    f = pl.kernel(
        _sc_body,
        out_type=(
            jax.ShapeDtypeStruct((B,), jnp.float32),
            jax.ShapeDtypeStruct((NEG, B), jnp.float32),
        ),
        mesh=mesh,
        compiler_params=pltpu.CompilerParams(
            use_tc_tiling_on_sc=False, needs_layout_passes=False
        ),
        scratch_types=[
            pltpu.VMEM((B_PER_W,), jnp.int32),
            pltpu.VMEM((B_PER_W,), jnp.int32),
            pltpu.VMEM((NEG, B_PER_W), jnp.int32),
            pltpu.VMEM((BLK, D), jnp.float32),
            pltpu.VMEM((NP, BLK, D), jnp.float32),
            pltpu.VMEM((NP, BLK), jnp.float32),
            pltpu.SemaphoreType.DMA,
        ],
    )
    return f(center_words, context_words, neg_T, in_emb, out_emb)


def _loss_body(pos_ref, neg_ref, out_ref):
    p = pos_ref[...]
    n = neg_ref[...]
    total = jnp.sum(jax.nn.log_sigmoid(p)) + jnp.sum(jax.nn.log_sigmoid(-n))
    out_ref[...] = jnp.reshape(-total / B, (1, 1))


@jax.jit
def _tc_loss(pos, neg):
    out = pl.pallas_call(
        _loss_body,
        out_shape=jax.ShapeDtypeStruct((1, 1), jnp.float32),
    )(pos.reshape(128, 128), neg.reshape(NEG * B // 128, 128))
    return out[0, 0]


def kernel(center_words, context_words, negative_words, in_emb, out_emb):
    neg_T = negative_words.T
    pos, negs = _sc_scores(center_words, context_words, neg_T, in_emb, out_emb)
    return _tc_loss(pos, negs)
